# baseline (speedup 1.0000x reference)
import functools

import jax
import jax.numpy as jnp
from jax import lax
from jax.experimental import pallas as pl
from jax.experimental.pallas import tpu as pltpu
from jax.experimental.pallas import tpu_sc as plsc

BATCH = 2
SEQ_LEN = 4096
STRUCT_LEN = 2048
D_MODEL = 768
TOKENS = BATCH * SEQ_LEN

NUM_CORES = 2
NUM_SUBCORES = 16
NUM_WORKERS = NUM_CORES * NUM_SUBCORES
TOK_PER_W = TOKENS // NUM_WORKERS
CHUNK = 64
NCHUNK = TOK_PER_W // CHUNK
STRUCT_WORKERS = STRUCT_LEN // TOK_PER_W
LANES = 16
VECS_PER_ROW = D_MODEL // LANES


def _body(pos_hbm, lab_hbm, lf_hbm, st_hbm, out_hbm,
          idx_v, sidx_v, rows_v, srows_v, sem_g, sem_s):
    wid = lax.axis_index("s") * NUM_CORES + lax.axis_index("c")
    base = wid * TOK_PER_W

    pltpu.sync_copy(pos_hbm.at[wid], idx_v)

    @pl.when(wid < STRUCT_WORKERS)
    def _():
        pltpu.sync_copy(lab_hbm.at[wid], sidx_v)

    for c in range(NCHUNK):
        pltpu.async_copy(lf_hbm.at[idx_v.at[c]], rows_v, sem_g).wait()

        @pl.when(wid < STRUCT_WORKERS)
        def _():
            pltpu.async_copy(st_hbm.at[sidx_v.at[c]], srows_v, sem_s).wait()

            def row_add(r, _):
                def vec_add(j, _):
                    x = srows_v[r, pl.ds(j * LANES, LANES)]
                    plsc.addupdate(rows_v.at[r, pl.ds(j * LANES, LANES)], x)
                    return 0
                lax.fori_loop(0, VECS_PER_ROW, vec_add, 0, unroll=8)
                return 0
            lax.fori_loop(0, CHUNK, row_add, 0)

        pltpu.sync_copy(rows_v, out_hbm.at[pl.ds(base + c * CHUNK, CHUNK)])


@jax.jit
def _run(pos, lab, lf_table, st_table):
    kern = functools.partial(
        pl.kernel,
        mesh=plsc.VectorSubcoreMesh(core_axis_name="c", subcore_axis_name="s"),
        out_type=jax.ShapeDtypeStruct((TOKENS, D_MODEL), jnp.float32),
        scratch_types=[
            pltpu.VMEM((NCHUNK, CHUNK), jnp.int32),
            pltpu.VMEM((NCHUNK, CHUNK), jnp.int32),
            pltpu.VMEM((CHUNK, D_MODEL), jnp.float32),
            pltpu.VMEM((CHUNK, D_MODEL), jnp.float32),
            pltpu.SemaphoreType.DMA,
            pltpu.SemaphoreType.DMA,
        ],
    )(_body)
    return kern(pos, lab, lf_table, st_table)


def kernel(positions, node_types_labels, longformer_table, struct_table):
    pos = positions.astype(jnp.int32).reshape(NUM_WORKERS, NCHUNK, CHUNK)
    lab = node_types_labels[0].astype(jnp.int32).reshape(
        STRUCT_WORKERS, NCHUNK, CHUNK)
    out = _run(pos, lab, longformer_table, struct_table)
    return out.reshape(BATCH, SEQ_LEN, D_MODEL)
